# baseline (speedup 1.0000x reference)
import jax
import jax.numpy as jnp
from jax.experimental import pallas as pl


def _trend_decomp_kernel(x_ref, seasonal_ref, trend_ref):
    b = pl.program_id(0)
    x = x_ref[0]
    s = x.shape[0]
    t = jax.lax.broadcasted_iota(jnp.int32, x.shape, 0).astype(jnp.float32)
    mean_t = (s - 1) * 0.5
    var_t = (s * s - 1.0) / 12.0
    inv_s = 1.0 / s
    mean_x = jnp.sum(x, axis=0, keepdims=True) * inv_s
    cov_xt = jnp.sum(x * (t - mean_t), axis=0, keepdims=True) * inv_s
    slope = cov_xt * (1.0 / var_t)
    intercept = mean_x - slope * mean_t
    trend_fit = slope * t + intercept
    seasonal = jnp.where(b == 0, jnp.zeros_like(x), x - trend_fit)
    seasonal_ref[0] = seasonal
    trend_ref[0] = x - seasonal


def kernel(x):
    B, S, C = x.shape
    out = jax.ShapeDtypeStruct((B, S, C), x.dtype)
    blk_c = 128
    spec = pl.BlockSpec((1, S, blk_c), lambda b, c: (b, 0, c))
    seasonal, trend = pl.pallas_call(
        _trend_decomp_kernel,
        grid=(B, C // blk_c),
        in_specs=[spec],
        out_specs=[spec, spec],
        out_shape=[out, out],
    )(x)
    return (seasonal, trend)


# speedup vs baseline: 1.4121x; 1.4121x over previous
import jax
import jax.numpy as jnp
from jax.experimental import pallas as pl


def _trend_decomp_kernel(x_ref, seasonal_ref, trend_ref):
    blk_b = x_ref.shape[0]
    base_b = pl.program_id(0) * blk_b
    x = x_ref[...]
    s = x.shape[1]
    t = jax.lax.broadcasted_iota(jnp.int32, x.shape, 1).astype(jnp.float32)
    mean_t = (s - 1) * 0.5
    var_t = (s * s - 1.0) / 12.0
    inv_s = 1.0 / s
    mean_x = jnp.sum(x, axis=1, keepdims=True) * inv_s
    cov_xt = jnp.sum(x * (t - mean_t), axis=1, keepdims=True) * inv_s
    slope = cov_xt * (1.0 / var_t)
    intercept = mean_x - slope * mean_t
    trend_fit = slope * t + intercept
    batch_ids = base_b + jax.lax.broadcasted_iota(jnp.int32, x.shape, 0)
    seasonal = jnp.where(batch_ids == 0, jnp.zeros_like(x), x - trend_fit)
    seasonal_ref[...] = seasonal
    trend_ref[...] = x - seasonal


def kernel(x):
    B, S, C = x.shape
    out = jax.ShapeDtypeStruct((B, S, C), x.dtype)
    blk_b = 2
    spec = pl.BlockSpec((blk_b, S, C), lambda b: (b, 0, 0))
    seasonal, trend = pl.pallas_call(
        _trend_decomp_kernel,
        grid=(B // blk_b,),
        in_specs=[spec],
        out_specs=[spec, spec],
        out_shape=[out, out],
    )(x)
    return (seasonal, trend)


# speedup vs baseline: 1.4134x; 1.0009x over previous
import jax
import jax.numpy as jnp
from jax.experimental import pallas as pl
from jax.experimental.pallas import tpu as pltpu


def _trend_decomp_kernel(x_ref, seasonal_ref, trend_ref):
    blk_b = x_ref.shape[0]
    base_b = pl.program_id(0) * blk_b
    x = x_ref[...]
    s = x.shape[1]
    t = jax.lax.broadcasted_iota(jnp.int32, x.shape, 1).astype(jnp.float32)
    mean_t = (s - 1) * 0.5
    var_t = (s * s - 1.0) / 12.0
    inv_s = 1.0 / s
    mean_x = jnp.sum(x, axis=1, keepdims=True) * inv_s
    cov_xt = jnp.sum(x * (t - mean_t), axis=1, keepdims=True) * inv_s
    slope = cov_xt * (1.0 / var_t)
    intercept = mean_x - slope * mean_t
    trend_fit = slope * t + intercept
    batch_ids = base_b + jax.lax.broadcasted_iota(jnp.int32, x.shape, 0)
    seasonal = jnp.where(batch_ids == 0, jnp.zeros_like(x), x - trend_fit)
    seasonal_ref[...] = seasonal
    trend_ref[...] = x - seasonal


def kernel(x):
    B, S, C = x.shape
    out = jax.ShapeDtypeStruct((B, S, C), x.dtype)
    blk_b = 2
    spec = pl.BlockSpec((blk_b, S, C), lambda b: (b, 0, 0))
    seasonal, trend = pl.pallas_call(
        _trend_decomp_kernel,
        grid=(B // blk_b,),
        in_specs=[spec],
        out_specs=[spec, spec],
        out_shape=[out, out],
        compiler_params=pltpu.CompilerParams(
            dimension_semantics=("parallel",),
        ),
    )(x)
    return (seasonal, trend)
